# baseline (speedup 1.0000x reference)
import jax
import jax.numpy as jnp
from jax.experimental import pallas as pl
from jax.experimental.pallas import tpu as pltpu

B, N, D = 64, 512, 128


def _gcn_body(x_ref, adj_ref, w0_ref, w1_ref, w2_ref, b0_ref, b1_ref, b2_ref,
              g_ref):
    adj = adj_ref[0].astype(jnp.bfloat16)
    h = x_ref[0]
    for w_ref, b_ref in ((w0_ref, b0_ref), (w1_ref, b1_ref), (w2_ref, b2_ref)):
        t = jnp.dot(h.astype(jnp.bfloat16), w_ref[...].astype(jnp.bfloat16),
                    preferred_element_type=jnp.float32)
        h = jax.nn.relu(jnp.dot(adj, t.astype(jnp.bfloat16),
                                preferred_element_type=jnp.float32)
                        + b_ref[...])
    g_ref[0] = jnp.sum(h, axis=0, keepdims=True)


def _head(g_ref, ro_w_ref, ro_b_ref, fc_w0_ref, fc_b0_ref, fc_w1_ref,
          fc_b1_ref, out_ref):
    g = jnp.dot(g_ref[...], ro_w_ref[...],
                preferred_element_type=jnp.float32) + ro_b_ref[...]
    g = jax.nn.relu(jnp.dot(g, fc_w0_ref[...],
                            preferred_element_type=jnp.float32) + fc_b0_ref[...])
    out_ref[...] = jax.nn.sigmoid(
        jnp.dot(g, fc_w1_ref[...], preferred_element_type=jnp.float32)
        + fc_b1_ref[...])


def kernel(x, adj, gnn_w0, gnn_b0, gnn_w1, gnn_b1, gnn_w2, gnn_b2,
           ro_w, ro_b, fc_w0, fc_b0, fc_w1, fc_b1):
    row = lambda v: v.reshape(1, -1).astype(jnp.float32)
    rep2 = lambda shape: pl.BlockSpec(shape, lambda b: (0, 0))

    g = pl.pallas_call(
        _gcn_body,
        grid=(B,),
        in_specs=[
            pl.BlockSpec((1, N, D), lambda b: (b, 0, 0)),
            pl.BlockSpec((1, N, N), lambda b: (b, 0, 0)),
            rep2((D, D)), rep2((D, D)), rep2((D, D)),
            rep2((1, D)), rep2((1, D)), rep2((1, D)),
        ],
        out_specs=pl.BlockSpec((1, 1, D), lambda b: (b, 0, 0)),
        out_shape=jax.ShapeDtypeStruct((B, 1, D), jnp.float32),
        compiler_params=pltpu.CompilerParams(
            dimension_semantics=("parallel",)),
    )(x, adj, gnn_w0, gnn_w1, gnn_w2, row(gnn_b0), row(gnn_b1), row(gnn_b2))

    fc_w1p = jnp.zeros((D, D), dtype=jnp.float32).at[:, 0].set(fc_w1[:, 0])
    fc_b1p = jnp.broadcast_to(row(fc_b1), (1, D))
    out = pl.pallas_call(
        _head,
        in_specs=[pl.BlockSpec((B, D), lambda: (0, 0)),
                  pl.BlockSpec((D, D), lambda: (0, 0)),
                  pl.BlockSpec((1, D), lambda: (0, 0)),
                  pl.BlockSpec((D, D), lambda: (0, 0)),
                  pl.BlockSpec((1, D), lambda: (0, 0)),
                  pl.BlockSpec((D, D), lambda: (0, 0)),
                  pl.BlockSpec((1, D), lambda: (0, 0))],
        out_specs=pl.BlockSpec((B, D), lambda: (0, 0)),
        out_shape=jax.ShapeDtypeStruct((B, D), jnp.float32),
    )(g[:, 0, :], ro_w, row(ro_b), fc_w0, row(fc_b0), fc_w1p, fc_b1p)
    return out[:, :1]


# speedup vs baseline: 1.2811x; 1.2811x over previous
import jax
import jax.numpy as jnp
from jax.experimental import pallas as pl
from jax.experimental.pallas import tpu as pltpu

B, N, D = 64, 512, 128
G = 4


def _gcn_body(x_ref, adj_ref, w0_ref, w1_ref, w2_ref, b0_ref, b1_ref, b2_ref,
              g_ref):
    for i in range(G):
        adj = adj_ref[i]
        h = x_ref[i]
        for w_ref, b_ref in ((w0_ref, b0_ref), (w1_ref, b1_ref),
                             (w2_ref, b2_ref)):
            t = jnp.dot(h, w_ref[...], preferred_element_type=jnp.float32)
            h = jax.nn.relu(
                jnp.dot(adj, t, preferred_element_type=jnp.float32)
                + b_ref[...])
        g_ref[i] = jnp.sum(h, axis=0, keepdims=True)


def _head(g_ref, ro_w_ref, ro_b_ref, fc_w0_ref, fc_b0_ref, fc_w1_ref,
          fc_b1_ref, out_ref):
    g = jnp.dot(g_ref[...], ro_w_ref[...],
                preferred_element_type=jnp.float32) + ro_b_ref[...]
    g = jax.nn.relu(jnp.dot(g, fc_w0_ref[...],
                            preferred_element_type=jnp.float32) + fc_b0_ref[...])
    out_ref[...] = jax.nn.sigmoid(
        jnp.dot(g, fc_w1_ref[...], preferred_element_type=jnp.float32)
        + fc_b1_ref[...])


def kernel(x, adj, gnn_w0, gnn_b0, gnn_w1, gnn_b1, gnn_w2, gnn_b2,
           ro_w, ro_b, fc_w0, fc_b0, fc_w1, fc_b1):
    row = lambda v: v.reshape(1, -1).astype(jnp.float32)
    rep2 = lambda shape: pl.BlockSpec(shape, lambda b: (0, 0))

    g = pl.pallas_call(
        _gcn_body,
        grid=(B // G,),
        in_specs=[
            pl.BlockSpec((G, N, D), lambda b: (b, 0, 0)),
            pl.BlockSpec((G, N, N), lambda b: (b, 0, 0)),
            rep2((D, D)), rep2((D, D)), rep2((D, D)),
            rep2((1, D)), rep2((1, D)), rep2((1, D)),
        ],
        out_specs=pl.BlockSpec((G, 1, D), lambda b: (b, 0, 0)),
        out_shape=jax.ShapeDtypeStruct((B, 1, D), jnp.float32),
        compiler_params=pltpu.CompilerParams(
            dimension_semantics=("parallel",)),
    )(x, adj, gnn_w0, gnn_w1, gnn_w2, row(gnn_b0), row(gnn_b1), row(gnn_b2))

    fc_w1p = jnp.zeros((D, D), dtype=jnp.float32).at[:, 0].set(fc_w1[:, 0])
    fc_b1p = jnp.broadcast_to(row(fc_b1), (1, D))
    out = pl.pallas_call(
        _head,
        in_specs=[pl.BlockSpec((B, D), lambda: (0, 0)),
                  pl.BlockSpec((D, D), lambda: (0, 0)),
                  pl.BlockSpec((1, D), lambda: (0, 0)),
                  pl.BlockSpec((D, D), lambda: (0, 0)),
                  pl.BlockSpec((1, D), lambda: (0, 0)),
                  pl.BlockSpec((D, D), lambda: (0, 0)),
                  pl.BlockSpec((1, D), lambda: (0, 0))],
        out_specs=pl.BlockSpec((B, D), lambda: (0, 0)),
        out_shape=jax.ShapeDtypeStruct((B, D), jnp.float32),
    )(g[:, 0, :], ro_w, row(ro_b), fc_w0, row(fc_b0), fc_w1p, fc_b1p)
    return out[:, :1]


# speedup vs baseline: 1.3480x; 1.0522x over previous
import jax
import jax.numpy as jnp
from jax.experimental import pallas as pl
from jax.experimental.pallas import tpu as pltpu

B, N, D = 64, 512, 128
G = 8


def _gcn_body(x_ref, adj_ref, w0_ref, w1_ref, w2_ref, b0_ref, b1_ref, b2_ref,
              g_ref):
    for i in range(G):
        adj = adj_ref[i]
        h = x_ref[i]
        for w_ref, b_ref in ((w0_ref, b0_ref), (w1_ref, b1_ref),
                             (w2_ref, b2_ref)):
            t = jnp.dot(h, w_ref[...], preferred_element_type=jnp.float32)
            h = jax.nn.relu(
                jnp.dot(adj, t, preferred_element_type=jnp.float32)
                + b_ref[...])
        g_ref[i] = jnp.sum(h, axis=0, keepdims=True)


def _head(g_ref, ro_w_ref, ro_b_ref, fc_w0_ref, fc_b0_ref, fc_w1_ref,
          fc_b1_ref, out_ref):
    g = jnp.dot(g_ref[...], ro_w_ref[...],
                preferred_element_type=jnp.float32) + ro_b_ref[...]
    g = jax.nn.relu(jnp.dot(g, fc_w0_ref[...],
                            preferred_element_type=jnp.float32) + fc_b0_ref[...])
    out_ref[...] = jax.nn.sigmoid(
        jnp.dot(g, fc_w1_ref[...], preferred_element_type=jnp.float32)
        + fc_b1_ref[...])


def kernel(x, adj, gnn_w0, gnn_b0, gnn_w1, gnn_b1, gnn_w2, gnn_b2,
           ro_w, ro_b, fc_w0, fc_b0, fc_w1, fc_b1):
    row = lambda v: v.reshape(1, -1).astype(jnp.float32)
    rep2 = lambda shape: pl.BlockSpec(shape, lambda b: (0, 0))

    g = pl.pallas_call(
        _gcn_body,
        grid=(B // G,),
        in_specs=[
            pl.BlockSpec((G, N, D), lambda b: (b, 0, 0)),
            pl.BlockSpec((G, N, N), lambda b: (b, 0, 0)),
            rep2((D, D)), rep2((D, D)), rep2((D, D)),
            rep2((1, D)), rep2((1, D)), rep2((1, D)),
        ],
        out_specs=pl.BlockSpec((G, 1, D), lambda b: (b, 0, 0)),
        out_shape=jax.ShapeDtypeStruct((B, 1, D), jnp.float32),
        compiler_params=pltpu.CompilerParams(
            dimension_semantics=("parallel",)),
    )(x, adj, gnn_w0, gnn_w1, gnn_w2, row(gnn_b0), row(gnn_b1), row(gnn_b2))

    fc_w1p = jnp.zeros((D, D), dtype=jnp.float32).at[:, 0].set(fc_w1[:, 0])
    fc_b1p = jnp.broadcast_to(row(fc_b1), (1, D))
    out = pl.pallas_call(
        _head,
        in_specs=[pl.BlockSpec((B, D), lambda: (0, 0)),
                  pl.BlockSpec((D, D), lambda: (0, 0)),
                  pl.BlockSpec((1, D), lambda: (0, 0)),
                  pl.BlockSpec((D, D), lambda: (0, 0)),
                  pl.BlockSpec((1, D), lambda: (0, 0)),
                  pl.BlockSpec((D, D), lambda: (0, 0)),
                  pl.BlockSpec((1, D), lambda: (0, 0))],
        out_specs=pl.BlockSpec((B, D), lambda: (0, 0)),
        out_shape=jax.ShapeDtypeStruct((B, D), jnp.float32),
    )(g[:, 0, :], ro_w, row(ro_b), fc_w0, row(fc_b0), fc_w1p, fc_b1p)
    return out[:, :1]
